# baseline (speedup 1.0000x reference)
import functools

import jax
import jax.numpy as jnp
from jax import lax
from jax.experimental import pallas as pl
from jax.experimental.pallas import tpu as pltpu
from jax.experimental.pallas import tpu_sc as plsc

L = 16
N = 1024
NCHUNK = N // L
G = 4
NC = 2
NS = 16
NW = NC * NS
NPAIR = 64
PPW = NPAIR // NW

_mesh = plsc.VectorSubcoreMesh(core_axis_name="c", subcore_axis_name="s")


@functools.partial(
    pl.kernel,
    mesh=_mesh,
    out_type=jax.ShapeDtypeStruct((NW, L), jnp.float32),
    scratch_types=[
        pltpu.VMEM((N,), jnp.float32),
        pltpu.VMEM((N,), jnp.float32),
        pltpu.VMEM((N,), jnp.float32),
        pltpu.VMEM((N,), jnp.float32),
        pltpu.VMEM((N,), jnp.float32),
        pltpu.VMEM((N,), jnp.float32),
        pltpu.VMEM((N,), jnp.float32),
        pltpu.VMEM((N,), jnp.float32),
        pltpu.VMEM((N,), jnp.float32),
        pltpu.VMEM((L,), jnp.float32),
    ],
)
def _chamfer_sc(x_hbm, y_hbm, out_hbm,
                xs0, xs1, xs2, xsq, ym0, ym1, ym2, ysq, colbuf, accbuf):
    wid = lax.axis_index("s") * NC + lax.axis_index("c")
    inf = jnp.full((L,), jnp.inf, jnp.float32)
    acc = jnp.zeros((L,), jnp.float32)
    lane_iota = lax.iota(jnp.int32, L)

    def lane_min(v):
        m = v
        for sh in (8, 4, 2, 1):
            m = jnp.minimum(m, m.at[lane_iota ^ sh].get(mode="promise_in_bounds"))
        return m

    for t in range(PPW):
        p = wid * PPW + t
        pltpu.sync_copy(x_hbm.at[0, p], xs0)
        pltpu.sync_copy(x_hbm.at[1, p], xs1)
        pltpu.sync_copy(x_hbm.at[2, p], xs2)
        pltpu.sync_copy(y_hbm.at[0, p], ym0)
        pltpu.sync_copy(y_hbm.at[1, p], ym1)
        pltpu.sync_copy(y_hbm.at[2, p], ym2)

        def prep(c, _):
            s = pl.ds(c * L, L)
            a0 = ym0[s]
            a1 = ym1[s]
            a2 = ym2[s]
            ysq[s] = a0 * a0 + a1 * a1 + a2 * a2
            ym0[s] = a0 * -2.0
            ym1[s] = a1 * -2.0
            ym2[s] = a2 * -2.0
            b0 = xs0[s]
            b1 = xs1[s]
            b2 = xs2[s]
            xsq[s] = b0 * b0 + b1 * b1 + b2 * b2
            colbuf[s] = inf
            return 0

        lax.fori_loop(0, NCHUNK, prep, 0)

        def rowchunk(rc, rowacc):
            rs = pl.ds(rc * L, L)
            xv0 = xs0[rs]
            xv1 = xs1[rs]
            xv2 = xs2[rs]
            xvq = xsq[rs]
            for sb in range(L // G):
                xb = []
                for g in range(G):
                    lane = sb * G + g
                    xb.append((jnp.full((L,), xv0[lane]),
                               jnp.full((L,), xv1[lane]),
                               jnp.full((L,), xv2[lane]),
                               jnp.full((L,), xvq[lane])))

                def chunk(c, rows):
                    s = pl.ds(c * L, L)
                    m0 = ym0[s]
                    m1 = ym1[s]
                    m2 = ym2[s]
                    ys = ysq[s]
                    cm = colbuf[s]
                    new_rows = []
                    for g in range(G):
                        e = (ys + xb[g][3]) + xb[g][0] * m0 + xb[g][1] * m1 + xb[g][2] * m2
                        new_rows.append(jnp.minimum(rows[g], e))
                        cm = jnp.minimum(cm, e)
                    colbuf[s] = cm
                    return tuple(new_rows)

                rows = lax.fori_loop(0, NCHUNK, chunk, (inf,) * G)
                for g in range(G):
                    rowacc = rowacc + lane_min(rows[g])
            return rowacc

        rowacc = lax.fori_loop(0, NCHUNK, rowchunk, jnp.zeros((L,), jnp.float32))

        def fin(c, a):
            s = pl.ds(c * L, L)
            return a + colbuf[s]

        pairvec = lax.fori_loop(0, NCHUNK, fin, jnp.zeros((L,), jnp.float32))
        pairvec = pairvec + rowacc * 0.0625
        w = jnp.where(p % 8 == 0, jnp.float32(2.0), jnp.float32(1.0))
        acc = acc + w * pairvec

    accbuf[...] = acc * 0.125
    pltpu.sync_copy(accbuf, out_hbm.at[wid])


def kernel(X_v, target_X_v):
    x = jnp.transpose(X_v.reshape(NPAIR, N, 3), (2, 0, 1))
    y = jnp.transpose(target_X_v.reshape(NPAIR, N, 3), (2, 0, 1))
    out = _chamfer_sc(x, y)
    return jnp.sum(out)


# speedup vs baseline: 1.5086x; 1.5086x over previous
import functools

import jax
import jax.numpy as jnp
from jax import lax
from jax.experimental import pallas as pl
from jax.experimental.pallas import tpu as pltpu

N = 1024
NPAIR = 64
DPAD = 8


def _tc_body(x_ref, y_ref, o_ref):
    p = pl.program_id(0)
    xb = x_ref[0]
    yb = y_ref[0]
    m = lax.dot_general(xb, yb, (((1,), (1,)), ((), ())),
                        preferred_element_type=jnp.float32)
    x2 = jnp.sum(xb * xb, axis=1)
    y2 = jnp.sum(yb * yb, axis=1)
    d = x2[:, None] + y2[None, :] - 2.0 * m
    total = jnp.sum(jnp.min(d, axis=1)) + jnp.sum(jnp.min(d, axis=0))
    w = jnp.where(p % 8 == 0, jnp.float32(2.0), jnp.float32(1.0))

    @pl.when(p == 0)
    def _():
        o_ref[0, 0] = jnp.float32(0.0)

    o_ref[0, 0] += w * total


_tc_call = pl.pallas_call(
    _tc_body,
    grid=(NPAIR,),
    in_specs=[
        pl.BlockSpec((1, N, DPAD), lambda p: (p, 0, 0)),
        pl.BlockSpec((1, N, DPAD), lambda p: (p, 0, 0)),
    ],
    out_specs=pl.BlockSpec(memory_space=pltpu.SMEM),
    out_shape=jax.ShapeDtypeStruct((1, 1), jnp.float32),
    compiler_params=pltpu.CompilerParams(
        dimension_semantics=("arbitrary",),
    ),
)


def kernel(X_v, target_X_v):
    x = jnp.pad(X_v.reshape(NPAIR, N, 3), ((0, 0), (0, 0), (0, DPAD - 3)))
    y = jnp.pad(target_X_v.reshape(NPAIR, N, 3), ((0, 0), (0, 0), (0, DPAD - 3)))
    out = _tc_call(x, y)
    return out[0, 0] * 0.125


# speedup vs baseline: 3.0069x; 1.9932x over previous
import functools

import jax
import jax.numpy as jnp
from jax import lax
from jax.experimental import pallas as pl
from jax.experimental.pallas import tpu as pltpu

N = 1024
NPAIR = 64
CW = 128


def _tc_body(x_ref, y_ref, o_ref):
    p = pl.program_id(0)
    xb = x_ref[0]
    yb = y_ref[0]
    x2 = jnp.sum(xb * xb, axis=0)
    y2 = jnp.sum(yb * yb, axis=0)
    x2col = x2[:, None]

    runmin = None
    colsum = jnp.float32(0.0)
    for c in range(N // CW):
        yc = yb[:, c * CW:(c + 1) * CW]
        mc = lax.dot_general(xb, yc, (((0,), (0,)), ((), ())),
                             preferred_element_type=jnp.float32)
        dc = (x2col - 2.0 * mc) + y2[c * CW:(c + 1) * CW][None, :]
        runmin = dc if c == 0 else jnp.minimum(runmin, dc)
        colsum = colsum + jnp.sum(jnp.min(dc, axis=0))
    rowsum = jnp.sum(jnp.min(runmin, axis=1))
    total = rowsum + colsum
    w = jnp.where(p % 8 == 0, jnp.float32(2.0), jnp.float32(1.0))

    @pl.when(p == 0)
    def _():
        o_ref[0, 0] = jnp.float32(0.0)

    o_ref[0, 0] += w * total


_tc_call = pl.pallas_call(
    _tc_body,
    grid=(NPAIR,),
    in_specs=[
        pl.BlockSpec((1, 3, N), lambda p: (p, 0, 0)),
        pl.BlockSpec((1, 3, N), lambda p: (p, 0, 0)),
    ],
    out_specs=pl.BlockSpec(memory_space=pltpu.SMEM),
    out_shape=jax.ShapeDtypeStruct((1, 1), jnp.float32),
    compiler_params=pltpu.CompilerParams(
        dimension_semantics=("arbitrary",),
    ),
)


def kernel(X_v, target_X_v):
    x = jnp.transpose(X_v.reshape(NPAIR, N, 3), (0, 2, 1))
    y = jnp.transpose(target_X_v.reshape(NPAIR, N, 3), (0, 2, 1))
    out = _tc_call(x, y)
    return out[0, 0] * 0.125


# speedup vs baseline: 3.5369x; 1.1762x over previous
import functools

import jax
import jax.numpy as jnp
from jax import lax
from jax.experimental import pallas as pl
from jax.experimental.pallas import tpu as pltpu

N = 1024
NPAIR = 64
CW = 128
KA = 8


def _tc_body(x_ref, y_ref, o_ref, xa, ya):
    p = pl.program_id(0)
    xb = x_ref[0]
    yb = y_ref[0]
    x2 = jnp.sum(xb * xb, axis=0)
    y2 = jnp.sum(yb * yb, axis=0)

    xa[0:3, :] = xb * -2.0
    xa[3:4, :] = x2[None, :]
    ya[0:3, :] = yb
    ya[4:5, :] = y2[None, :]

    @pl.when(p == 0)
    def _():
        xa[4:5, :] = jnp.ones((1, N), jnp.float32)
        xa[5:8, :] = jnp.zeros((3, N), jnp.float32)
        ya[3:4, :] = jnp.ones((1, N), jnp.float32)
        ya[5:8, :] = jnp.zeros((3, N), jnp.float32)
        o_ref[0, 0] = jnp.float32(0.0)

    xav = xa[...]
    yav = ya[...]
    runmin = None
    colsum = jnp.float32(0.0)
    for c in range(N // CW):
        yc = yav[:, c * CW:(c + 1) * CW]
        dc = lax.dot_general(xav, yc, (((0,), (0,)), ((), ())),
                             preferred_element_type=jnp.float32)
        runmin = dc if c == 0 else jnp.minimum(runmin, dc)
        colsum = colsum + jnp.sum(jnp.min(dc, axis=0))
    rowsum = jnp.sum(jnp.min(runmin, axis=1))
    total = rowsum + colsum
    w = jnp.where(p % 8 == 0, jnp.float32(2.0), jnp.float32(1.0))
    o_ref[0, 0] += w * total


_tc_call = pl.pallas_call(
    _tc_body,
    grid=(NPAIR,),
    in_specs=[
        pl.BlockSpec((1, 3, N), lambda p: (p, 0, 0)),
        pl.BlockSpec((1, 3, N), lambda p: (p, 0, 0)),
    ],
    out_specs=pl.BlockSpec(memory_space=pltpu.SMEM),
    out_shape=jax.ShapeDtypeStruct((1, 1), jnp.float32),
    scratch_shapes=[
        pltpu.VMEM((KA, N), jnp.float32),
        pltpu.VMEM((KA, N), jnp.float32),
    ],
    compiler_params=pltpu.CompilerParams(
        dimension_semantics=("arbitrary",),
    ),
)


def kernel(X_v, target_X_v):
    x = jnp.transpose(X_v.reshape(NPAIR, N, 3), (0, 2, 1))
    y = jnp.transpose(target_X_v.reshape(NPAIR, N, 3), (0, 2, 1))
    out = _tc_call(x, y)
    return out[0, 0] * 0.125


# speedup vs baseline: 5.0257x; 1.4209x over previous
import functools

import jax
import jax.numpy as jnp
from jax import lax
from jax.experimental import pallas as pl
from jax.experimental.pallas import tpu as pltpu

N = 1024
NPAIR = 64
CW = 128
KA = 8
P = 4


def _tc_body(x_ref, y_ref, o_ref, xa, ya):
    s = pl.program_id(0)

    @pl.when(s == 0)
    def _():
        xa[4:5, :] = jnp.ones((1, N), jnp.float32)
        xa[5:8, :] = jnp.zeros((3, N), jnp.float32)
        ya[3:4, :] = jnp.ones((1, N), jnp.float32)
        ya[5:8, :] = jnp.zeros((3, N), jnp.float32)
        o_ref[0, 0] = jnp.float32(0.0)

    acc = jnp.float32(0.0)
    for q in range(P):
        xb = x_ref[q]
        yb = y_ref[q]
        x2 = jnp.sum(xb * xb, axis=0)
        y2 = jnp.sum(yb * yb, axis=0)

        xa[0:3, :] = xb * -2.0
        xa[3:4, :] = x2[None, :]
        ya[0:3, :] = yb
        ya[4:5, :] = y2[None, :]

        xav = xa[...]
        yav = ya[...]
        runmin = None
        colsum = jnp.float32(0.0)
        for c in range(N // CW):
            yc = yav[:, c * CW:(c + 1) * CW]
            dc = lax.dot_general(xav, yc, (((0,), (0,)), ((), ())),
                                 preferred_element_type=jnp.float32)
            runmin = dc if c == 0 else jnp.minimum(runmin, dc)
            colsum = colsum + jnp.sum(jnp.min(dc, axis=0))
        rowsum = jnp.sum(jnp.min(runmin, axis=1))
        total = rowsum + colsum
        pid = s * P + q
        w = jnp.where(pid % 8 == 0, jnp.float32(2.0), jnp.float32(1.0))
        acc = acc + w * total

    o_ref[0, 0] += acc


_tc_call = pl.pallas_call(
    _tc_body,
    grid=(NPAIR // P,),
    in_specs=[
        pl.BlockSpec((P, 3, N), lambda s: (s, 0, 0)),
        pl.BlockSpec((P, 3, N), lambda s: (s, 0, 0)),
    ],
    out_specs=pl.BlockSpec(memory_space=pltpu.SMEM),
    out_shape=jax.ShapeDtypeStruct((1, 1), jnp.float32),
    scratch_shapes=[
        pltpu.VMEM((KA, N), jnp.float32),
        pltpu.VMEM((KA, N), jnp.float32),
    ],
    compiler_params=pltpu.CompilerParams(
        dimension_semantics=("arbitrary",),
    ),
)


def kernel(X_v, target_X_v):
    x = jnp.transpose(X_v.reshape(NPAIR, N, 3), (0, 2, 1))
    y = jnp.transpose(target_X_v.reshape(NPAIR, N, 3), (0, 2, 1))
    out = _tc_call(x, y)
    return out[0, 0] * 0.125
